# baseline (speedup 1.0000x reference)
import functools

import jax
import jax.numpy as jnp
from jax import lax
from jax.experimental import pallas as pl
from jax.experimental.pallas import tpu as pltpu
from jax.experimental.pallas import tpu_sc as plsc

E = 16
TOP_K = 2
D_MODEL = 1024
D_FF = 768
SEQ = 2048

BLK = 64
N_ASSIGN = TOP_K * SEQ
N_PAD = N_ASSIGN + E * BLK
N_BLOCKS = N_PAD // BLK

NC = 2
NS = 16
NW = NC * NS
NSLICE = 2
S_ROWS = N_PAD // NSLICE
S_BLOCKS = S_ROWS // BLK
G_ROWS = S_ROWS // NW
G_CHUNK = 40
G_NCH = G_ROWS // G_CHUNK
C_TOK = SEQ // NW
C_CHUNK = 32
LANES = 16


def _cumsum0(x, n):
    s = 1
    while s < n:
        x = x + jnp.pad(x, ((s, 0), (0, 0)))[:n]
        s *= 2
    return x


def _router_plan_body(x_ref, w_ref, d1_ref, d2_ref, m1_ref, m2_ref, bg_ref):
    x = x_ref[...]
    w = w_ref[...]
    logits = lax.dot_general(x, w, (((1,), (1,)), ((), ())),
                             preferred_element_type=jnp.float32)
    m = jnp.max(logits, axis=-1, keepdims=True)
    ex = jnp.exp(logits - m)
    p = ex / jnp.sum(ex, axis=-1, keepdims=True)
    iota = lax.broadcasted_iota(jnp.int32, (SEQ, E), 1)
    m1 = jnp.max(p, axis=-1, keepdims=True)
    i1 = jnp.min(jnp.where(p == m1, iota, E), axis=-1)
    p2 = jnp.where(iota == i1[:, None], -jnp.inf, p)
    m2 = jnp.max(p2, axis=-1, keepdims=True)
    i2 = jnp.min(jnp.where(p2 == m2, iota, E), axis=-1)

    oh1 = (iota == i1[:, None]).astype(jnp.float32)
    oh2 = (iota == i2[:, None]).astype(jnp.float32)
    oh = jnp.concatenate([oh1, oh2], axis=0)
    csum = _cumsum0(oh, N_ASSIGN)
    counts = csum[N_ASSIGN - 1:N_ASSIGN, :]
    pc = jnp.floor((counts + (BLK - 1)) / BLK) * BLK
    tri = (lax.broadcasted_iota(jnp.int32, (E, E), 0)
           <= lax.broadcasted_iota(jnp.int32, (E, E), 1)).astype(jnp.float32)
    cum = lax.dot_general(pc, tri, (((1,), (0,)), ((), ())),
                          preferred_element_type=jnp.float32)
    pad_off = cum - pc
    rank = jnp.sum((csum - 1.0) * oh, axis=1)
    base = jnp.sum(oh * pad_off, axis=1)
    dest = (base + rank).astype(jnp.int32)
    d1_ref[...] = dest[:SEQ]
    d2_ref[...] = dest[SEQ:]
    m1_ref[...] = m1.reshape(SEQ)
    m2_ref[...] = m2.reshape(SEQ)
    blk_start = (lax.broadcasted_iota(jnp.int32, (N_BLOCKS, E), 0)
                 .astype(jnp.float32) * jnp.float32(BLK))
    bg = jnp.sum((blk_start >= cum).astype(jnp.int32), axis=1)
    bg_ref[...] = jnp.minimum(bg, E - 1)


def _router_plan(hf, router_w):
    return pl.pallas_call(
        _router_plan_body,
        out_shape=(
            jax.ShapeDtypeStruct((SEQ,), jnp.int32),
            jax.ShapeDtypeStruct((SEQ,), jnp.int32),
            jax.ShapeDtypeStruct((SEQ,), jnp.float32),
            jax.ShapeDtypeStruct((SEQ,), jnp.float32),
            jax.ShapeDtypeStruct((N_BLOCKS,), jnp.int32),
        ),
    )(hf, router_w)


def _sc_gather_body(hf_hbm, tok_hbm, out_hbm,
                    idx_v, b0, b1, gs0, gs1, ws0, ws1):
    wid = lax.axis_index("s") * NC + lax.axis_index("c")
    base = wid * G_ROWS
    bufs = (b0, b1)
    gsems = (gs0, gs1)
    wsems = (ws0, ws1)
    pltpu.sync_copy(tok_hbm.at[pl.ds(base, G_ROWS)], idx_v)
    gd = {}
    wb = {}

    def start_gather(c):
        gd[c] = pltpu.async_copy(
            hf_hbm.at[idx_v.at[pl.ds(c * G_CHUNK, G_CHUNK)]],
            bufs[c % 2], gsems[c % 2])

    start_gather(0)
    for c in range(G_NCH):
        if c + 1 < G_NCH:
            if c >= 1:
                wb[c - 1].wait()
            start_gather(c + 1)
        gd[c].wait()
        wb[c] = pltpu.async_copy(
            bufs[c % 2], out_hbm.at[pl.ds(base + c * G_CHUNK, G_CHUNK)],
            wsems[c % 2])
    if G_NCH >= 2:
        wb[G_NCH - 2].wait()
    wb[G_NCH - 1].wait()


def _sc_gather(hf, tok_slice):
    return pl.kernel(
        _sc_gather_body,
        out_type=jax.ShapeDtypeStruct((S_ROWS, D_MODEL), jnp.float32),
        mesh=plsc.VectorSubcoreMesh(core_axis_name="c", subcore_axis_name="s"),
        scratch_types=[
            pltpu.VMEM((G_ROWS,), jnp.int32),
            pltpu.VMEM((G_CHUNK, D_MODEL), jnp.float32),
            pltpu.VMEM((G_CHUNK, D_MODEL), jnp.float32),
            pltpu.SemaphoreType.DMA,
            pltpu.SemaphoreType.DMA,
            pltpu.SemaphoreType.DMA,
            pltpu.SemaphoreType.DMA,
        ],
    )(hf, tok_slice)


def _ffn_body(bg_ref, x_ref, wt_ref, g_ref, u_ref, d_ref, y_ref):
    del bg_ref
    x = x_ref[...]
    g = lax.dot_general(x, g_ref[0], (((1,), (1,)), ((), ())),
                        preferred_element_type=jnp.float32)
    u = lax.dot_general(x, u_ref[0], (((1,), (1,)), ((), ())),
                        preferred_element_type=jnp.float32)
    h = (g * jax.nn.sigmoid(g)) * u
    y = lax.dot_general(h, d_ref[0], (((1,), (1,)), ((), ())),
                        preferred_element_type=jnp.float32)
    y_ref[...] = y * wt_ref[...]


def _ffn_body_aliased(bg_ref, x_ref, wt_ref, g_ref, u_ref, d_ref, yp_ref,
                      y_ref):
    del yp_ref
    _ffn_body(bg_ref, x_ref, wt_ref, g_ref, u_ref, d_ref, y_ref)


def _ffn_grid_spec(k, aliased):
    in_specs = [
        pl.BlockSpec((BLK, D_MODEL), lambda i, bg: (i, 0)),
        pl.BlockSpec((BLK, 1), lambda i, bg: (i, 0)),
        pl.BlockSpec((1, D_FF, D_MODEL), lambda i, bg: (bg[i], 0, 0)),
        pl.BlockSpec((1, D_FF, D_MODEL), lambda i, bg: (bg[i], 0, 0)),
        pl.BlockSpec((1, D_MODEL, D_FF), lambda i, bg: (bg[i], 0, 0)),
    ]
    if aliased:
        in_specs.append(pl.BlockSpec((BLK, D_MODEL), lambda i, bg: (0, 0)))
    return pltpu.PrefetchScalarGridSpec(
        num_scalar_prefetch=1,
        grid=(S_BLOCKS,),
        in_specs=in_specs,
        out_specs=pl.BlockSpec((BLK, D_MODEL),
                               lambda i, bg: (k * S_BLOCKS + i, 0)),
    )


def _ffn_slice(k, bg_s, x_s, w_s, gate_proj, up_proj, down_proj, y_prev):
    out_shape = jax.ShapeDtypeStruct((N_PAD, D_MODEL), jnp.float32)
    if y_prev is None:
        return pl.pallas_call(
            _ffn_body,
            grid_spec=_ffn_grid_spec(k, aliased=False),
            out_shape=out_shape,
        )(bg_s, x_s, w_s, gate_proj, up_proj, down_proj)
    return pl.pallas_call(
        _ffn_body_aliased,
        grid_spec=_ffn_grid_spec(k, aliased=True),
        out_shape=out_shape,
        input_output_aliases={6: 0},
    )(bg_s, x_s, w_s, gate_proj, up_proj, down_proj, y_prev)


def _sc_combine_body(y_hbm, p0_hbm, p1_hbm, out_hbm,
                     i0_v, i1_v, b0_v, b1_v, sem0, sem1):
    wid = lax.axis_index("s") * NC + lax.axis_index("c")
    for c in range(C_TOK // C_CHUNK):
        base = wid * C_TOK + c * C_CHUNK
        pltpu.sync_copy(p0_hbm.at[pl.ds(base, C_CHUNK)], i0_v)
        pltpu.sync_copy(p1_hbm.at[pl.ds(base, C_CHUNK)], i1_v)
        cp0 = pltpu.async_copy(y_hbm.at[i0_v], b0_v, sem0)
        cp1 = pltpu.async_copy(y_hbm.at[i1_v], b1_v, sem1)
        cp0.wait()
        cp1.wait()

        def add_row(r, carry):
            for k in range(D_MODEL // LANES):
                sl = pl.ds(k * LANES, LANES)
                b0_v[r, sl] = b0_v[r, sl] + b1_v[r, sl]
            return carry

        lax.fori_loop(0, C_CHUNK, add_row, 0)
        pltpu.sync_copy(b0_v, out_hbm.at[pl.ds(base, C_CHUNK)])


def _sc_combine(y, p0, p1):
    return pl.kernel(
        _sc_combine_body,
        out_type=jax.ShapeDtypeStruct((SEQ, D_MODEL), jnp.float32),
        mesh=plsc.VectorSubcoreMesh(core_axis_name="c", subcore_axis_name="s"),
        scratch_types=[
            pltpu.VMEM((C_CHUNK,), jnp.int32),
            pltpu.VMEM((C_CHUNK,), jnp.int32),
            pltpu.VMEM((C_CHUNK, D_MODEL), jnp.float32),
            pltpu.VMEM((C_CHUNK, D_MODEL), jnp.float32),
            pltpu.SemaphoreType.DMA,
            pltpu.SemaphoreType.DMA,
        ],
    )(y, p0, p1)


def _dispatch_scatter(p0, p1, m1, m2):
    dest = jnp.concatenate([p0, p1])
    tok_f = jnp.arange(SEQ, dtype=jnp.float32)
    vals = jnp.stack([jnp.concatenate([tok_f, tok_f]),
                      jnp.concatenate([m1, m2])], axis=1)
    pad_tok = (jnp.arange(N_PAD, dtype=jnp.float32)
               % jnp.float32(SEQ))[:, None]
    init = jnp.concatenate([pad_tok, jnp.zeros((N_PAD, 1), jnp.float32)], 1)
    packed = init.at[dest].set(vals)
    token_pad = packed[:, 0].astype(jnp.int32)
    w_pad = packed[:, 1:2]
    return token_pad, w_pad


def kernel(hidden_states, router_w, gate_proj, up_proj, down_proj):
    b, s, dm = hidden_states.shape
    hf = hidden_states.reshape(s, dm)
    p0, p1, m1, m2, bg = _router_plan(hf, router_w)
    token_pad, w_pad = _dispatch_scatter(p0, p1, m1, m2)
    y = None
    for k in range(NSLICE):
        tok_k = lax.slice(token_pad, (k * S_ROWS,), ((k + 1) * S_ROWS,))
        x_k = _sc_gather(hf, tok_k)
        y = _ffn_slice(
            k,
            lax.slice(bg, (k * S_BLOCKS,), ((k + 1) * S_BLOCKS,)),
            x_k,
            lax.slice(w_pad, (k * S_ROWS, 0), ((k + 1) * S_ROWS, 1)),
            gate_proj, up_proj, down_proj, y)
    out = _sc_combine(y, p0, p1)
    return out.reshape(b, s, dm)


# speedup vs baseline: 1.2521x; 1.2521x over previous
import functools

import jax
import jax.numpy as jnp
from jax import lax
from jax.experimental import pallas as pl
from jax.experimental.pallas import tpu as pltpu
from jax.experimental.pallas import tpu_sc as plsc

E = 16
TOP_K = 2
D_MODEL = 1024
D_FF = 768
SEQ = 2048

BLK = 128
N_ASSIGN = TOP_K * SEQ
N_PAD = N_ASSIGN + E * BLK
N_BLOCKS = N_PAD // BLK

NC = 2
NS = 16
NW = NC * NS
A_TILE = N_ASSIGN // NW
A_CHUNK = 32
A_NCH = A_TILE // A_CHUNK
C_TOK = SEQ // NW
C_CHUNK = 32
LANES = 16


def _cumsum0(x, n):
    s = 1
    while s < n:
        x = x + jnp.pad(x, ((s, 0), (0, 0)))[:n]
        s *= 2
    return x


def _router_plan_body(x_ref, w_ref, d1_ref, d2_ref, m1_ref, m2_ref, bg_ref):
    x = x_ref[...]
    w = w_ref[...]
    logits = lax.dot_general(x, w, (((1,), (1,)), ((), ())),
                             preferred_element_type=jnp.float32)
    m = jnp.max(logits, axis=-1, keepdims=True)
    ex = jnp.exp(logits - m)
    p = ex / jnp.sum(ex, axis=-1, keepdims=True)
    iota = lax.broadcasted_iota(jnp.int32, (SEQ, E), 1)
    m1 = jnp.max(p, axis=-1, keepdims=True)
    i1 = jnp.min(jnp.where(p == m1, iota, E), axis=-1)
    p2 = jnp.where(iota == i1[:, None], -jnp.inf, p)
    m2 = jnp.max(p2, axis=-1, keepdims=True)
    i2 = jnp.min(jnp.where(p2 == m2, iota, E), axis=-1)

    oh1 = (iota == i1[:, None]).astype(jnp.float32)
    oh2 = (iota == i2[:, None]).astype(jnp.float32)
    oh = jnp.concatenate([oh1, oh2], axis=0)
    csum = _cumsum0(oh, N_ASSIGN)
    counts = csum[N_ASSIGN - 1:N_ASSIGN, :]
    pc = jnp.floor((counts + (BLK - 1)) / BLK) * BLK
    tri = (lax.broadcasted_iota(jnp.int32, (E, E), 0)
           <= lax.broadcasted_iota(jnp.int32, (E, E), 1)).astype(jnp.float32)
    cum = lax.dot_general(pc, tri, (((1,), (0,)), ((), ())),
                          preferred_element_type=jnp.float32)
    pad_off = cum - pc
    rank = jnp.sum((csum - 1.0) * oh, axis=1)
    base = jnp.sum(oh * pad_off, axis=1)
    dest = (base + rank).astype(jnp.int32)
    d1_ref[...] = dest[:SEQ]
    d2_ref[...] = dest[SEQ:]
    m1_ref[...] = m1.reshape(SEQ)
    m2_ref[...] = m2.reshape(SEQ)
    blk_start = (lax.broadcasted_iota(jnp.int32, (N_BLOCKS, E), 0)
                 .astype(jnp.float32) * jnp.float32(BLK))
    bg = jnp.sum((blk_start >= cum).astype(jnp.int32), axis=1)
    bg_ref[...] = jnp.minimum(bg, E - 1)


def _router_plan(hf, router_w):
    return pl.pallas_call(
        _router_plan_body,
        out_shape=(
            jax.ShapeDtypeStruct((SEQ,), jnp.int32),
            jax.ShapeDtypeStruct((SEQ,), jnp.int32),
            jax.ShapeDtypeStruct((SEQ,), jnp.float32),
            jax.ShapeDtypeStruct((SEQ,), jnp.float32),
            jax.ShapeDtypeStruct((N_BLOCKS,), jnp.int32),
        ),
    )(hf, router_w)


def _sc_scatter_body(hf_hbm, pcat_hbm, x_hbm,
                     i0, i1, b0, b1, rs0, rs1, ws0, ws1):
    wid = lax.axis_index("s") * NC + lax.axis_index("c")
    a_base = wid * A_TILE
    row_base = lax.rem(a_base, SEQ)
    idxs = (i0, i1)
    bufs = (b0, b1)
    rsems = (rs0, rs1)
    wsems = (ws0, ws1)
    rd = {}
    wr = {}

    def start_read(c):
        pltpu.sync_copy(pcat_hbm.at[pl.ds(a_base + c * A_CHUNK, A_CHUNK)],
                        idxs[c % 2])
        rd[c] = pltpu.async_copy(
            hf_hbm.at[pl.ds(row_base + c * A_CHUNK, A_CHUNK)],
            bufs[c % 2], rsems[c % 2])

    start_read(0)
    for c in range(A_NCH):
        rd[c].wait()
        wr[c] = pltpu.async_copy(bufs[c % 2], x_hbm.at[idxs[c % 2]],
                                 wsems[c % 2])
        if c + 1 < A_NCH:
            if c >= 1:
                wr[c - 1].wait()
            start_read(c + 1)
    wr[A_NCH - 2].wait()
    wr[A_NCH - 1].wait()


def _sc_scatter(hf, pcat):
    return pl.kernel(
        _sc_scatter_body,
        out_type=jax.ShapeDtypeStruct((N_PAD, D_MODEL), jnp.float32),
        mesh=plsc.VectorSubcoreMesh(core_axis_name="c", subcore_axis_name="s"),
        scratch_types=[
            pltpu.VMEM((A_CHUNK,), jnp.int32),
            pltpu.VMEM((A_CHUNK,), jnp.int32),
            pltpu.VMEM((A_CHUNK, D_MODEL), jnp.float32),
            pltpu.VMEM((A_CHUNK, D_MODEL), jnp.float32),
            pltpu.SemaphoreType.DMA,
            pltpu.SemaphoreType.DMA,
            pltpu.SemaphoreType.DMA,
            pltpu.SemaphoreType.DMA,
        ],
    )(hf, pcat)


def _ffn_body(bg_ref, x_ref, g_ref, u_ref, d_ref, y_ref):
    del bg_ref
    x = x_ref[...]
    g = lax.dot_general(x, g_ref[0], (((1,), (1,)), ((), ())),
                        preferred_element_type=jnp.float32)
    u = lax.dot_general(x, u_ref[0], (((1,), (1,)), ((), ())),
                        preferred_element_type=jnp.float32)
    h = (g * jax.nn.sigmoid(g)) * u
    y_ref[...] = lax.dot_general(h, d_ref[0], (((1,), (1,)), ((), ())),
                                 preferred_element_type=jnp.float32)


def _ffn_grid_spec():
    return pltpu.PrefetchScalarGridSpec(
        num_scalar_prefetch=1,
        grid=(N_BLOCKS,),
        in_specs=[
            pl.BlockSpec((BLK, D_MODEL), lambda i, bg: (i, 0)),
            pl.BlockSpec((1, D_FF, D_MODEL), lambda i, bg: (bg[i], 0, 0)),
            pl.BlockSpec((1, D_FF, D_MODEL), lambda i, bg: (bg[i], 0, 0)),
            pl.BlockSpec((1, D_MODEL, D_FF), lambda i, bg: (bg[i], 0, 0)),
        ],
        out_specs=pl.BlockSpec((BLK, D_MODEL), lambda i, bg: (i, 0)),
    )


def _ffn(bg, x_sorted, gate_proj, up_proj, down_proj):
    return pl.pallas_call(
        _ffn_body,
        grid_spec=_ffn_grid_spec(),
        out_shape=jax.ShapeDtypeStruct((N_PAD, D_MODEL), jnp.float32),
    )(bg, x_sorted, gate_proj, up_proj, down_proj)


def _sc_combine_body(y_hbm, p0_hbm, p1_hbm, m1_hbm, m2_hbm, out_hbm,
                     i0_v, i1_v, w0_v, w1_v, b0_v, b1_v, sem0, sem1):
    wid = lax.axis_index("s") * NC + lax.axis_index("c")
    for c in range(C_TOK // C_CHUNK):
        base = wid * C_TOK + c * C_CHUNK
        pltpu.sync_copy(p0_hbm.at[pl.ds(base, C_CHUNK)], i0_v)
        pltpu.sync_copy(p1_hbm.at[pl.ds(base, C_CHUNK)], i1_v)
        pltpu.sync_copy(m1_hbm.at[pl.ds(base, C_CHUNK)], w0_v)
        pltpu.sync_copy(m2_hbm.at[pl.ds(base, C_CHUNK)], w1_v)
        cp0 = pltpu.async_copy(y_hbm.at[i0_v], b0_v, sem0)
        cp1 = pltpu.async_copy(y_hbm.at[i1_v], b1_v, sem1)
        cp0.wait()
        cp1.wait()

        def add_row(r, carry):
            w0 = w0_v[r, :]
            w1 = w1_v[r, :]
            for k in range(D_MODEL // LANES):
                sl = pl.ds(k * LANES, LANES)
                b0_v[r, sl] = b0_v[r, sl] * w0 + b1_v[r, sl] * w1
            return carry

        lax.fori_loop(0, C_CHUNK, add_row, 0)
        pltpu.sync_copy(b0_v, out_hbm.at[pl.ds(base, C_CHUNK)])


def _sc_combine(y, p0, p1, m1, m2):
    m1b = jnp.broadcast_to(m1[:, None], (SEQ, LANES))
    m2b = jnp.broadcast_to(m2[:, None], (SEQ, LANES))
    return pl.kernel(
        _sc_combine_body,
        out_type=jax.ShapeDtypeStruct((SEQ, D_MODEL), jnp.float32),
        mesh=plsc.VectorSubcoreMesh(core_axis_name="c", subcore_axis_name="s"),
        scratch_types=[
            pltpu.VMEM((C_CHUNK,), jnp.int32),
            pltpu.VMEM((C_CHUNK,), jnp.int32),
            pltpu.VMEM((C_CHUNK, LANES), jnp.float32),
            pltpu.VMEM((C_CHUNK, LANES), jnp.float32),
            pltpu.VMEM((C_CHUNK, D_MODEL), jnp.float32),
            pltpu.VMEM((C_CHUNK, D_MODEL), jnp.float32),
            pltpu.SemaphoreType.DMA,
            pltpu.SemaphoreType.DMA,
        ],
    )(y, p0, p1, m1b, m2b)


def kernel(hidden_states, router_w, gate_proj, up_proj, down_proj):
    b, s, dm = hidden_states.shape
    hf = hidden_states.reshape(s, dm)
    p0, p1, m1, m2, bg = _router_plan(hf, router_w)
    pcat = jnp.concatenate([p0, p1])
    x_sorted = _sc_scatter(hf, pcat)
    y = _ffn(bg, x_sorted, gate_proj, up_proj, down_proj)
    out = _sc_combine(y, p0, p1, m1, m2)
    return out.reshape(b, s, dm)
